# baseline (speedup 1.0000x reference)
import functools

import jax
import jax.numpy as jnp
from jax import lax
from jax.experimental import pallas as pl
from jax.experimental.pallas import tpu as pltpu


def _fused_body(x_ref, w_ref, b_ref, s_ref, o_ref, scr, *, eps, cin, cout,
                wp, hh, grp):
    xb = x_ref[0]
    c, h2, w2 = xb.shape
    inv_c = 1.0 / cin
    s1 = jnp.sum(xb, axis=0)
    s2 = jnp.sum(xb * xb, axis=0)
    mu = s1 * inv_c
    var = jnp.maximum(s2 * inv_c - mu * mu, 0.0)
    r = lax.rsqrt(var + eps)
    xn = (xb - mu[None]) * r[None]
    scr[...] = xn.reshape(c * h2, w2)

    sel = s_ref[...]
    bias = b_ref[...]
    for g in range(hh // grp):
        us = ([], [])
        for i in range(grp):
            ho = g * grp + i
            slab = scr[pl.ds(ho, c, h2), :]
            y = lax.dot_general(
                slab.astype(jnp.bfloat16), sel, (((1,), (0,)), ((), ())),
                preferred_element_type=jnp.float32)
            yb = y.astype(jnp.bfloat16)
            for kw in range(2):
                us[kw].append(jnp.concatenate(
                    [yb[:, kw * wp:(kw + 1) * wp],
                     yb[:, (2 + kw) * wp:(3 + kw) * wp]], axis=0))
        acc = bias
        for kw in range(2):
            u4 = jnp.concatenate(us[kw], axis=1)
            acc = acc + lax.dot_general(
                w_ref[kw], u4, (((1,), (0,)), ((), ())),
                preferred_element_type=jnp.float32)
        o_ref[0, :, pl.ds(g * grp * wp, grp * wp)] = acc.astype(o_ref.dtype)


def kernel(x, ln_gamma, ln_beta, conv_w, conv_b, *, eps=1e-6):
    B, C, H, W = x.shape
    Cout = conv_w.shape[0]
    Hh, Wh = H // 2, W // 2
    Wp = 32
    GRP = 4

    wmat = jnp.transpose(conv_w, (2, 3, 1, 0)).reshape(4, C, Cout)
    wmat = wmat.astype(jnp.float32)
    wmat_f = wmat * ln_gamma.astype(jnp.float32)[None, :, None]
    bias_f = conv_b.astype(jnp.float32) + jnp.einsum(
        "c,jco->o", ln_beta.astype(jnp.float32), wmat)
    wT = jnp.transpose(wmat_f, (0, 2, 1))
    wK = jnp.stack([jnp.concatenate([wT[kw], wT[2 + kw]], axis=1)
                    for kw in range(2)]).astype(jnp.bfloat16)
    bias2d = jnp.broadcast_to(bias_f[:, None], (Cout, GRP * Wp))
    li = lax.broadcasted_iota(jnp.int32, (2 * W, 4 * Wp), 0)
    ci = lax.broadcasted_iota(jnp.int32, (2 * W, 4 * Wp), 1)
    jt, wo = ci // Wp, ci % Wp
    kh, kw = jt // 2, jt % 2
    sel = ((li == kh * W + 2 * wo + kw) & (wo < Wh)).astype(jnp.bfloat16)

    body = functools.partial(_fused_body, eps=eps, cin=C, cout=Cout,
                             wp=Wp, hh=Hh, grp=GRP)
    x2 = x.reshape(B, C, Hh, 2 * W)

    def _call(single_buffer):
        wkw = dict(pipeline_mode=pl.Buffered(1)) if single_buffer else {}
        return pl.pallas_call(
            body,
            out_shape=jax.ShapeDtypeStruct((B, Cout, Hh * Wp), x.dtype),
            grid=(B,),
            in_specs=[
                pl.BlockSpec((1, C, Hh, 2 * W), lambda b: (b, 0, 0, 0)),
                pl.BlockSpec((2, Cout, 2 * C), lambda b: (0, 0, 0), **wkw),
                pl.BlockSpec((Cout, GRP * Wp), lambda b: (0, 0), **wkw),
                pl.BlockSpec((2 * W, 4 * Wp), lambda b: (0, 0), **wkw),
            ],
            out_specs=pl.BlockSpec((1, Cout, Hh * Wp), lambda b: (b, 0, 0)),
            scratch_shapes=[pltpu.VMEM((C * Hh, 2 * W), jnp.float32)],
            compiler_params=pltpu.CompilerParams(
                dimension_semantics=("parallel",),
                vmem_limit_bytes=64 * 1024 * 1024),
            cost_estimate=pl.CostEstimate(
                flops=int(2 * B * Hh * Wp * 4 * C * Cout
                          + 8 * B * H * W * C),
                transcendentals=int(B * H * W),
                bytes_accessed=int(x.size * 4 + B * Cout * Hh * Wp * 4)),
        )(x2, wK, bias2d, sel)

    try:
        out1 = _call(True)
    except Exception:
        out1 = _call(False)
    return out1.reshape(B, Cout, Hh, Wp)[:, :, :, :Wh]


# speedup vs baseline: 1.1492x; 1.1492x over previous
import functools

import jax
import jax.numpy as jnp
from jax import lax
from jax.experimental import pallas as pl
from jax.experimental.pallas import tpu as pltpu


def _fused_body(x_ref, w_ref, b_ref, s_ref, o_ref, scr, *, eps, cin, cout,
                wp, hh, grp):
    xb = x_ref[0]
    c, hw = xb.shape
    w2 = hw // hh
    inv_c = 1.0 / cin
    s1 = jnp.sum(xb, axis=0, keepdims=True)
    s2 = jnp.sum(xb * xb, axis=0, keepdims=True)
    mu = s1 * inv_c
    var = jnp.maximum(s2 * inv_c - mu * mu, 0.0)
    r = lax.rsqrt(var + eps)
    scr[...] = (xb - mu) * r

    sel = s_ref[...]
    bias = b_ref[...]
    for g in range(hh // grp):
        us = ([], [])
        for i in range(grp):
            ho = g * grp + i
            slab = scr[:, pl.ds(ho * w2, w2)]
            y = lax.dot_general(
                slab.astype(jnp.bfloat16), sel, (((1,), (0,)), ((), ())),
                preferred_element_type=jnp.float32)
            yb = y.astype(jnp.bfloat16)
            for kw in range(2):
                us[kw].append(jnp.concatenate(
                    [yb[:, kw * wp:(kw + 1) * wp],
                     yb[:, (2 + kw) * wp:(3 + kw) * wp]], axis=0))
        acc = bias
        for kw in range(2):
            u4 = jnp.concatenate(us[kw], axis=1)
            acc = acc + lax.dot_general(
                w_ref[kw], u4, (((1,), (0,)), ((), ())),
                preferred_element_type=jnp.float32)
        o_ref[0, :, pl.ds(g * grp * wp, grp * wp)] = acc.astype(o_ref.dtype)


def kernel(x, ln_gamma, ln_beta, conv_w, conv_b, *, eps=1e-6):
    B, C, H, W = x.shape
    Cout = conv_w.shape[0]
    Hh, Wh = H // 2, W // 2
    Wp = 32
    GRP = 4

    wmat = jnp.transpose(conv_w, (2, 3, 1, 0)).reshape(4, C, Cout)
    wmat = wmat.astype(jnp.float32)
    wmat_f = wmat * ln_gamma.astype(jnp.float32)[None, :, None]
    bias_f = conv_b.astype(jnp.float32) + jnp.einsum(
        "c,jco->o", ln_beta.astype(jnp.float32), wmat)
    wT = jnp.transpose(wmat_f, (0, 2, 1))
    wK = jnp.stack([jnp.concatenate([wT[kw], wT[2 + kw]], axis=1)
                    for kw in range(2)]).astype(jnp.bfloat16)
    bias2d = jnp.broadcast_to(bias_f[:, None], (Cout, GRP * Wp))
    li = lax.broadcasted_iota(jnp.int32, (2 * W, 4 * Wp), 0)
    ci = lax.broadcasted_iota(jnp.int32, (2 * W, 4 * Wp), 1)
    jt, wo = ci // Wp, ci % Wp
    kh, kw = jt // 2, jt % 2
    sel = ((li == kh * W + 2 * wo + kw) & (wo < Wh)).astype(jnp.bfloat16)

    body = functools.partial(_fused_body, eps=eps, cin=C, cout=Cout,
                             wp=Wp, hh=Hh, grp=GRP)
    x2 = x.reshape(B, C, H * W)

    def _call(single_buffer):
        wkw = dict(pipeline_mode=pl.Buffered(1)) if single_buffer else {}
        return pl.pallas_call(
            body,
            out_shape=jax.ShapeDtypeStruct((B, Cout, Hh * Wp), x.dtype),
            grid=(B,),
            in_specs=[
                pl.BlockSpec((1, C, H * W), lambda b: (b, 0, 0)),
                pl.BlockSpec((2, Cout, 2 * C), lambda b: (0, 0, 0), **wkw),
                pl.BlockSpec((Cout, GRP * Wp), lambda b: (0, 0), **wkw),
                pl.BlockSpec((2 * W, 4 * Wp), lambda b: (0, 0), **wkw),
            ],
            out_specs=pl.BlockSpec((1, Cout, Hh * Wp), lambda b: (b, 0, 0)),
            scratch_shapes=[pltpu.VMEM((C, H * W), jnp.float32)],
            compiler_params=pltpu.CompilerParams(
                dimension_semantics=("parallel",),
                vmem_limit_bytes=64 * 1024 * 1024),
            cost_estimate=pl.CostEstimate(
                flops=int(2 * B * Hh * Wp * 4 * C * Cout
                          + 8 * B * H * W * C),
                transcendentals=int(B * H * W),
                bytes_accessed=int(x.size * 4 + B * Cout * Hh * Wp * 4)),
        )(x2, wK, bias2d, sel)

    try:
        out1 = _call(True)
    except Exception:
        out1 = _call(False)
    return out1.reshape(B, Cout, Hh, Wp)[:, :, :, :Wh]
